# Initial kernel scaffold; baseline (speedup 1.0000x reference)
import jax
import jax.numpy as jnp
from jax.experimental import pallas as pl


def kernel(p, params):
    raise NotImplementedError("write your pallas kernel here")


# speedup vs baseline: 6.0498x; 6.0498x over previous
import functools

import jax
import jax.numpy as jnp
from jax import lax
from jax.experimental import pallas as pl
from jax.experimental.pallas import tpu as pltpu
from jax.experimental.pallas import tpu_sc as plsc

F32 = jnp.float32
BF16 = jnp.bfloat16
HS, HV = 128, 16
KNN = 16
NBLK = 256
CVEC = HS + 3 * HV
CPAD = 256
RBF_N = 62
RBF_STEP = 20.0 / 61.0
RBF_COEFF = -0.5 / RBF_STEP ** 2
EPS = 1e-8


def _b16(x):
    return x.astype(BF16).astype(F32)


def _knn_body(pblk_ref, pT_ref, idx_ref, dx_ref, dy_ref, dz_ref):
    b = pl.program_id(0)
    i = pl.program_id(1)
    pblk = pblk_ref[0]
    pT = pT_ref[0]
    n = pT.shape[1]

    dot = jnp.dot(pblk[:, 0:3], pT[0:3], preferred_element_type=F32)
    sq_j = pT[0:1] * pT[0:1] + pT[1:2] * pT[1:2] + pT[2:3] * pT[2:3]
    sq_i = (pblk[:, 0:1] * pblk[:, 0:1] + pblk[:, 1:2] * pblk[:, 1:2]
            + pblk[:, 2:3] * pblk[:, 2:3])
    d2 = jnp.maximum((sq_i + sq_j) - 2.0 * dot, 0.0)
    col = lax.broadcasted_iota(jnp.int32, (NBLK, n), 1)
    row = i * NBLK + lax.broadcasted_iota(jnp.int32, (NBLK, n), 0)
    d2 = jnp.where(col == row, d2 + 1e10, d2)

    ddx = pT[0:1] - pblk[:, 0:1]
    ddy = pT[1:2] - pblk[:, 1:2]
    ddz = pT[2:3] - pblk[:, 2:3]

    for k in range(KNN):
        mn = jnp.min(d2, axis=1, keepdims=True)
        cand = jnp.where(d2 == mn, col, n)
        amin = jnp.min(cand, axis=1, keepdims=True)
        onehot = col == amin
        idx_ref[0, :, k:k + 1] = amin + b * n
        dx_ref[0, :, k:k + 1] = jnp.sum(jnp.where(onehot, ddx, 0.0),
                                        axis=1, keepdims=True)
        dy_ref[0, :, k:k + 1] = jnp.sum(jnp.where(onehot, ddy, 0.0),
                                        axis=1, keepdims=True)
        dz_ref[0, :, k:k + 1] = jnp.sum(jnp.where(onehot, ddz, 0.0),
                                        axis=1, keepdims=True)
        d2 = jnp.where(onehot, 1e30, d2)


def _knn(pz, pT):
    B, N, _ = pz.shape
    grid = (B, N // NBLK)
    io = jax.ShapeDtypeStruct((B, N, KNN), jnp.int32)
    fo = jax.ShapeDtypeStruct((B, N, KNN), F32)
    blk_nk = pl.BlockSpec((1, NBLK, KNN), lambda b, i: (b, i, 0))
    return pl.pallas_call(
        _knn_body,
        grid=grid,
        in_specs=[
            pl.BlockSpec((1, NBLK, 8), lambda b, i: (b, i, 0)),
            pl.BlockSpec((1, 8, N), lambda b, i: (b, 0, 0)),
        ],
        out_specs=[blk_nk, blk_nk, blk_nk, blk_nk],
        out_shape=[io, fo, fo, fo],
    )(pz, pT)


def _sc_gather(table, idx):
    E = idx.shape[0]
    C = table.shape[1]
    info = plsc.get_sparse_core_info()
    nw = info.num_cores * info.num_subcores
    per_w = E // nw
    chunk = 128
    n_ch = per_w // chunk
    mesh = plsc.VectorSubcoreMesh(core_axis_name="c", subcore_axis_name="s")

    @functools.partial(
        pl.kernel,
        mesh=mesh,
        out_type=jax.ShapeDtypeStruct((E, C), F32),
        scratch_types=[
            pltpu.VMEM((chunk,), jnp.int32),
            pltpu.VMEM((chunk, C), F32),
            pltpu.SemaphoreType.DMA,
        ],
    )
    def gather_k(table_hbm, idx_hbm, out_hbm, idx_v, rows_v, sem):
        wid = lax.axis_index("s") * info.num_cores + lax.axis_index("c")
        base = wid * per_w

        def body(ci, carry):
            off = base + ci * chunk
            pltpu.sync_copy(idx_hbm.at[pl.ds(off, chunk)], idx_v)
            pltpu.async_copy(table_hbm.at[idx_v], rows_v, sem).wait()
            pltpu.sync_copy(rows_v, out_hbm.at[pl.ds(off, chunk)])
            return carry

        lax.fori_loop(0, n_ch, body, 0)

    return gather_k(table, idx)


def _gvp(s, vx, vy, vz, wh, wss, wsvn, bs, wv, act):
    hx = jnp.dot(vx, wh, preferred_element_type=F32)
    hy = jnp.dot(vy, wh, preferred_element_type=F32)
    hz = jnp.dot(vz, wh, preferred_element_type=F32)
    vn = jnp.sqrt(hx * hx + hy * hy + hz * hz + EPS)
    s = (jnp.dot(s, wss, preferred_element_type=F32)
         + jnp.dot(vn, wsvn, preferred_element_type=F32) + bs)
    ux = jnp.dot(hx, wv, preferred_element_type=F32)
    uy = jnp.dot(hy, wv, preferred_element_type=F32)
    uz = jnp.dot(hz, wv, preferred_element_type=F32)
    if act:
        s = jnp.maximum(s, 0.0)
        nrm = jnp.sqrt(ux * ux + uy * uy + uz * uz + EPS)
        g = jax.nn.sigmoid(nrm)
        ux, uy, uz = ux * g, uy * g, uz * g
    return s, ux, uy, uz


def _edge_body(has_node, nblk, refs):
    if has_node:
        (dxe, dye, dze, svi, svj,
         wh0, wssj, wses, wsov, wssi, wsvn0, bs0, wv0,
         wh1, wss1, wsvn1, bs1, wv1,
         wh2, wss2, wsvn2, bs2, wv2, out_ref) = refs
    else:
        (dxe, dye, dze,
         whe, wses, wsov, wsvn0, bs0, wv0,
         wh1, wss1, wsvn1, bs1, wv1,
         wh2, wss2, wsvn2, bs2, wv2, out_ref) = refs

    nbk = nblk * KNN
    dx = dxe[0]
    dy = dye[0]
    dz = dze[0]
    dist = jnp.sqrt(dx * dx + dy * dy + dz * dz + EPS)
    evx, evy, evz = dx / dist, dy / dist, dz / dist
    d10 = dist * 10.0
    offs = lax.broadcasted_iota(jnp.int32, (1, RBF_N), 1).astype(F32) * RBF_STEP
    diff = d10 - offs
    es = jnp.exp(RBF_COEFF * diff * diff)
    over = (d10 >= 20.0).astype(F32)

    ms = (jnp.dot(es, wses[...], preferred_element_type=F32)
          + over * wsov[...] + bs0[...])

    if has_node:
        svjv = svj[...]
        sviv = svi[...]
        sj = svjv[:, 0:HS]
        vjx = svjv[:, HS:HS + HV]
        vjy = svjv[:, HS + HV:HS + 2 * HV]
        vjz = svjv[:, HS + 2 * HV:CVEC]
        si = sviv[:, 0:HS]
        vix = sviv[:, HS:HS + HV]
        viy = sviv[:, HS + HV:HS + 2 * HV]
        viz = sviv[:, HS + 2 * HV:CVEC]

        def brd(t):
            w = t.shape[1]
            return jnp.broadcast_to(t[:, None, :], (nblk, KNN, w)).reshape(nbk, w)

        w33 = wh0[...]
        cx = jnp.concatenate([vjx, evx, brd(vix)], axis=1)
        cy = jnp.concatenate([vjy, evy, brd(viy)], axis=1)
        cz = jnp.concatenate([vjz, evz, brd(viz)], axis=1)
        hx = jnp.dot(cx, w33, preferred_element_type=F32)
        hy = jnp.dot(cy, w33, preferred_element_type=F32)
        hz = jnp.dot(cz, w33, preferred_element_type=F32)
        ms = ms + jnp.dot(sj, wssj[...], preferred_element_type=F32) \
                + brd(jnp.dot(si, wssi[...], preferred_element_type=F32))
    else:
        web = _b16(whe[...])
        hx = _b16(evx) * web
        hy = _b16(evy) * web
        hz = _b16(evz) * web

    vn = jnp.sqrt(hx * hx + hy * hy + hz * hz + EPS)
    ms = ms + jnp.dot(vn, wsvn0[...], preferred_element_type=F32)
    ms = jnp.maximum(ms, 0.0)
    ux = jnp.dot(hx, wv0[...], preferred_element_type=F32)
    uy = jnp.dot(hy, wv0[...], preferred_element_type=F32)
    uz = jnp.dot(hz, wv0[...], preferred_element_type=F32)
    nrm = jnp.sqrt(ux * ux + uy * uy + uz * uz + EPS)
    g = jax.nn.sigmoid(nrm)
    ux, uy, uz = ux * g, uy * g, uz * g

    ms, ux, uy, uz = _gvp(ms, ux, uy, uz, wh1[...], wss1[...], wsvn1[...],
                          bs1[...], wv1[...], act=True)
    ms, ux, uy, uz = _gvp(ms, ux, uy, uz, wh2[...], wss2[...], wsvn2[...],
                          bs2[...], wv2[...], act=False)

    def mean_k(a):
        w = a.shape[1]
        return jnp.mean(a.reshape(nblk, KNN, w), axis=1)

    def mean_k_lin(a):
        w = a.shape[1]
        a3 = a.reshape(nblk, KNN, w)
        acc = a3[:, 0, :]
        for k in range(1, KNN):
            acc = acc + a3[:, k, :]
        return acc / 16.0

    out_ref[:, 0:HS] = mean_k(ms)
    out_ref[:, HS:HS + HV] = mean_k_lin(ux)
    out_ref[:, HS + HV:HS + 2 * HV] = mean_k_lin(uy)
    out_ref[:, HS + 2 * HV:CVEC] = mean_k_lin(uz)
    out_ref[:, CVEC:] = jnp.zeros((nblk, CPAD - CVEC), F32)


def _full_spec(arr):
    return pl.BlockSpec(arr.shape, lambda b, i: (0,) * arr.ndim)


def _edge_call(dxe, dye, dze, svi, svj, w):
    B, NK, _ = dxe.shape
    N = NK // KNN
    grid = (B, N // NBLK)
    nbk = NBLK * KNN
    e_spec = pl.BlockSpec((1, nbk, 1), lambda b, i: (b, i, 0))
    node_spec = pl.BlockSpec((NBLK, CPAD),
                             lambda b, i, n=N: (b * (n // NBLK) + i, 0))
    edge_spec = pl.BlockSpec((nbk, CPAD),
                             lambda b, i, n=N: (b * (n // NBLK) + i, 0))
    has_node = svi is not None
    args = [dxe, dye, dze]
    specs = [e_spec, e_spec, e_spec]
    if has_node:
        args += [svi, svj]
        specs += [node_spec, edge_spec]
    args += list(w)
    specs += [_full_spec(a) for a in w]
    body = functools.partial(_edge_body, has_node, NBLK)

    def wrapped(*refs):
        body(refs)

    return pl.pallas_call(
        wrapped,
        grid=grid,
        in_specs=specs,
        out_specs=node_spec,
        out_shape=jax.ShapeDtypeStruct((B * N, CPAD), F32),
    )(*args)


def _head_body(sv_ref, wh_ref, wss_ref, wsvn_ref, bs_ref, wv_ref,
               ys_ref, r_ref):
    sv = sv_ref[...]
    s = sv[:, 0:HS]
    vx = sv[:, HS:HS + HV]
    vy = sv[:, HS + HV:HS + 2 * HV]
    vz = sv[:, HS + 2 * HV:CVEC]
    wh = wh_ref[...]
    hx = jnp.dot(vx, wh, preferred_element_type=F32)
    hy = jnp.dot(vy, wh, preferred_element_type=F32)
    hz = jnp.dot(vz, wh, preferred_element_type=F32)
    vn = jnp.sqrt(hx * hx + hy * hy + hz * hz + EPS)
    ys_ref[...] = (jnp.dot(s, wss_ref[...], preferred_element_type=F32)
                   + jnp.dot(vn, wsvn_ref[...], preferred_element_type=F32)
                   + bs_ref[...])
    wv = wv_ref[...]
    ux = jnp.dot(hx, wv, preferred_element_type=F32)
    uy = jnp.dot(hy, wv, preferred_element_type=F32)
    uz = jnp.dot(hz, wv, preferred_element_type=F32)
    v1x, v2x = ux[:, 0:1], ux[:, 1:2]
    v1y, v2y = uy[:, 0:1], uy[:, 1:2]
    v1z, v2z = uz[:, 0:1], uz[:, 1:2]
    n1 = jnp.sqrt(v1x * v1x + v1y * v1y + v1z * v1z + EPS)
    e1x, e1y, e1z = v1x / n1, v1y / n1, v1z / n1
    dt = e1x * v2x + e1y * v2y + e1z * v2z
    u2x, u2y, u2z = v2x - dt * e1x, v2y - dt * e1y, v2z - dt * e1z
    n2 = jnp.sqrt(u2x * u2x + u2y * u2y + u2z * u2z + EPS)
    e2x, e2y, e2z = u2x / n2, u2y / n2, u2z / n2
    e3x = e1y * e2z - e1z * e2y
    e3y = e1z * e2x - e1x * e2z
    e3z = e1x * e2y - e1y * e2x
    r_ref[:, 0:1] = e1x
    r_ref[:, 1:2] = e2x
    r_ref[:, 2:3] = e3x
    r_ref[:, 3:4] = e1y
    r_ref[:, 4:5] = e2y
    r_ref[:, 5:6] = e3y
    r_ref[:, 6:7] = e1z
    r_ref[:, 7:8] = e2z
    r_ref[:, 8:9] = e3z
    r_ref[:, 9:16] = jnp.zeros_like(sv[:, 0:7])


def _head_call(sv, w):
    M = sv.shape[0]
    grid = (1, M // NBLK)
    node_spec = pl.BlockSpec((NBLK, CPAD), lambda b, i: (i, 0))
    return pl.pallas_call(
        _head_body,
        grid=grid,
        in_specs=[node_spec] + [_full_spec(a) for a in w],
        out_specs=[pl.BlockSpec((NBLK, HS), lambda b, i: (i, 0)),
                   pl.BlockSpec((NBLK, 16), lambda b, i: (i, 0))],
        out_shape=[jax.ShapeDtypeStruct((M, HS), F32),
                   jax.ShapeDtypeStruct((M, 16), F32)],
    )(sv, *w)


def _prep_layer0(prm, has_node):
    wh, ws, bs, wv = prm['Wh'], prm['Ws'], prm['bs'], prm['Wv']
    out = []
    if has_node:
        out += [wh]
    else:
        out += [wh[16:17]]
    if has_node:
        out += [ws[0:128]]
    out += [ws[129:191], ws[191:192]]
    if has_node:
        out += [ws[192:320]]
    out += [ws[320:353], bs[None], wv]
    return out


def _prep_layer(prm):
    return [prm['Wh'], prm['Ws'][0:HS], prm['Ws'][HS:HS + HV],
            prm['bs'][None], prm['Wv']]


def kernel(p, params):
    B, N, _ = p.shape
    pz = jnp.concatenate([p, jnp.zeros((B, N, 5), F32)], axis=-1)
    pT = jnp.transpose(pz, (0, 2, 1))
    idx, dx, dy, dz = _knn(pz, pT)
    idxf = idx.reshape(B * N * KNN)
    dxe = dx.reshape(B, N * KNN, 1)
    dye = dy.reshape(B, N * KNN, 1)
    dze = dz.reshape(B, N * KNN, 1)

    sv = None
    for g in range(3):
        layers = params['gconv%d' % g]
        has_node = g > 0
        w = (_prep_layer0(layers[0], has_node)
             + _prep_layer(layers[1]) + _prep_layer(layers[2]))
        if has_node:
            svj = _sc_gather(sv, idxf)
            sv = _edge_call(dxe, dye, dze, sv, svj, w)
        else:
            sv = _edge_call(dxe, dye, dze, None, None, w)

    ow = params['out']
    wlist = [ow['Wh'], ow['Ws'][0:HS], ow['Ws'][HS:HS + HV],
             ow['bs'][None], ow['Wv']]
    ys, r16 = _head_call(sv, wlist)
    R = r16[:, :9].reshape(B, N, 3, 3)
    return R, ys.reshape(B, N, HS)


# speedup vs baseline: 6.0590x; 1.0015x over previous
import functools

import jax
import jax.numpy as jnp
from jax import lax
from jax.experimental import pallas as pl
from jax.experimental.pallas import tpu as pltpu
from jax.experimental.pallas import tpu_sc as plsc

F32 = jnp.float32
BF16 = jnp.bfloat16
HS, HV = 128, 16
KNN = 16
NBLK = 256
CVEC = HS + 3 * HV
CPAD = 256
RBF_N = 62
RBF_STEP = 20.0 / 61.0
RBF_COEFF = -0.5 / RBF_STEP ** 2
EPS = 1e-8


def _b16(x):
    return x.astype(BF16).astype(F32)


def _knn_body(pblk_ref, pT_ref, psp_holder_ref, idx_ref, dx_ref, dy_ref, dz_ref):
    b = pl.program_id(0)
    i = pl.program_id(1)
    pblk = pblk_ref[0]
    pT = pT_ref[0]
    n = pT.shape[1]

    dot = jnp.dot(pblk[:, 0:3], pT[0:3], preferred_element_type=F32)
    sq_j = pT[0:1] * pT[0:1] + pT[1:2] * pT[1:2] + pT[2:3] * pT[2:3]
    sq_i = (pblk[:, 0:1] * pblk[:, 0:1] + pblk[:, 1:2] * pblk[:, 1:2]
            + pblk[:, 2:3] * pblk[:, 2:3])
    d2 = jnp.maximum((sq_i + sq_j) - 2.0 * dot, 0.0)
    col = lax.broadcasted_iota(jnp.int32, (NBLK, n), 1)
    row = i * NBLK + lax.broadcasted_iota(jnp.int32, (NBLK, n), 0)
    d2 = jnp.where(col == row, d2 + 1e10, d2)

    del psp_holder_ref
    ddx = pT[0:1] - pblk[:, 0:1]
    ddy = pT[1:2] - pblk[:, 1:2]
    ddz = pT[2:3] - pblk[:, 2:3]

    for k in range(KNN):
        mn = jnp.min(d2, axis=1, keepdims=True)
        cand = jnp.where(d2 == mn, col, n)
        amin = jnp.min(cand, axis=1, keepdims=True)
        onehot = cand == amin
        idx_ref[0, :, k:k + 1] = amin + b * n
        dx_ref[0, :, k:k + 1] = jnp.sum(jnp.where(onehot, ddx, 0.0),
                                        axis=1, keepdims=True)
        dy_ref[0, :, k:k + 1] = jnp.sum(jnp.where(onehot, ddy, 0.0),
                                        axis=1, keepdims=True)
        dz_ref[0, :, k:k + 1] = jnp.sum(jnp.where(onehot, ddz, 0.0),
                                        axis=1, keepdims=True)
        d2 = jnp.where(onehot, 1e30, d2)


def _knn(pz, pT, psp):
    B, N, _ = pz.shape
    grid = (B, N // NBLK)
    io = jax.ShapeDtypeStruct((B, N, KNN), jnp.int32)
    fo = jax.ShapeDtypeStruct((B, N, KNN), F32)
    blk_nk = pl.BlockSpec((1, NBLK, KNN), lambda b, i: (b, i, 0))
    return pl.pallas_call(
        _knn_body,
        grid=grid,
        in_specs=[
            pl.BlockSpec((1, NBLK, 8), lambda b, i: (b, i, 0)),
            pl.BlockSpec((1, 8, N), lambda b, i: (b, 0, 0)),
            pl.BlockSpec((1, N, 24), lambda b, i: (b, 0, 0)),
        ],
        out_specs=[blk_nk, blk_nk, blk_nk, blk_nk],
        out_shape=[io, fo, fo, fo],
    )(pz, pT, psp)


def _sc_gather(table, idx):
    E = idx.shape[0]
    C = table.shape[1]
    info = plsc.get_sparse_core_info()
    nw = info.num_cores * info.num_subcores
    per_w = E // nw
    chunk = 128
    n_ch = per_w // chunk
    mesh = plsc.VectorSubcoreMesh(core_axis_name="c", subcore_axis_name="s")

    @functools.partial(
        pl.kernel,
        mesh=mesh,
        out_type=jax.ShapeDtypeStruct((E, C), F32),
        scratch_types=[
            pltpu.VMEM((chunk,), jnp.int32),
            pltpu.VMEM((chunk, C), F32),
            pltpu.SemaphoreType.DMA,
        ],
    )
    def gather_k(table_hbm, idx_hbm, out_hbm, idx_v, rows_v, sem):
        wid = lax.axis_index("s") * info.num_cores + lax.axis_index("c")
        base = wid * per_w

        def body(ci, carry):
            off = base + ci * chunk
            pltpu.sync_copy(idx_hbm.at[pl.ds(off, chunk)], idx_v)
            pltpu.async_copy(table_hbm.at[idx_v], rows_v, sem).wait()
            pltpu.sync_copy(rows_v, out_hbm.at[pl.ds(off, chunk)])
            return carry

        lax.fori_loop(0, n_ch, body, 0)

    return gather_k(table, idx)


def _gvp(s, vx, vy, vz, wh, wss, wsvn, bs, wv, act):
    hx = jnp.dot(vx, wh, preferred_element_type=F32)
    hy = jnp.dot(vy, wh, preferred_element_type=F32)
    hz = jnp.dot(vz, wh, preferred_element_type=F32)
    vn = jnp.sqrt(hx * hx + hy * hy + hz * hz + EPS)
    s = (jnp.dot(s, wss, preferred_element_type=F32)
         + jnp.dot(vn, wsvn, preferred_element_type=F32) + bs)
    ux = jnp.dot(hx, wv, preferred_element_type=F32)
    uy = jnp.dot(hy, wv, preferred_element_type=F32)
    uz = jnp.dot(hz, wv, preferred_element_type=F32)
    if act:
        s = jnp.maximum(s, 0.0)
        nrm = jnp.sqrt(ux * ux + uy * uy + uz * uz + EPS)
        g = jax.nn.sigmoid(nrm)
        ux, uy, uz = ux * g, uy * g, uz * g
    return s, ux, uy, uz


def _edge_body(has_node, nblk, refs):
    if has_node:
        (dxe, dye, dze, svi, svj,
         wh0, wssj, wses, wsov, wssi, wsvn0, bs0, wv0,
         wh1, wss1, wsvn1, bs1, wv1,
         wh2, wss2, wsvn2, bs2, wv2, out_ref) = refs
    else:
        (dxe, dye, dze,
         whe, wses, wsov, wsvn0, bs0, wv0,
         wh1, wss1, wsvn1, bs1, wv1,
         wh2, wss2, wsvn2, bs2, wv2, out_ref) = refs

    nbk = nblk * KNN
    dx = dxe[0]
    dy = dye[0]
    dz = dze[0]
    dist = jnp.sqrt(dx * dx + dy * dy + dz * dz + EPS)
    evx, evy, evz = dx / dist, dy / dist, dz / dist
    d10 = dist * 10.0
    offs = lax.broadcasted_iota(jnp.int32, (1, RBF_N), 1).astype(F32) * RBF_STEP
    diff = d10 - offs
    es = jnp.exp(RBF_COEFF * diff * diff)
    over = (d10 >= 20.0).astype(F32)

    ms = (jnp.dot(es, wses[...], preferred_element_type=F32)
          + over * wsov[...] + bs0[...])

    if has_node:
        svjv = svj[...]
        sviv = svi[...]
        sj = svjv[:, 0:HS]
        vjx = svjv[:, HS:HS + HV]
        vjy = svjv[:, HS + HV:HS + 2 * HV]
        vjz = svjv[:, HS + 2 * HV:CVEC]
        si = sviv[:, 0:HS]
        vix = sviv[:, HS:HS + HV]
        viy = sviv[:, HS + HV:HS + 2 * HV]
        viz = sviv[:, HS + 2 * HV:CVEC]

        def brd(t):
            w = t.shape[1]
            return jnp.broadcast_to(t[:, None, :], (nblk, KNN, w)).reshape(nbk, w)

        w33 = wh0[...]
        cx = jnp.concatenate([vjx, evx, brd(vix)], axis=1)
        cy = jnp.concatenate([vjy, evy, brd(viy)], axis=1)
        cz = jnp.concatenate([vjz, evz, brd(viz)], axis=1)
        hx = jnp.dot(cx, w33, preferred_element_type=F32)
        hy = jnp.dot(cy, w33, preferred_element_type=F32)
        hz = jnp.dot(cz, w33, preferred_element_type=F32)
        ms = ms + jnp.dot(sj, wssj[...], preferred_element_type=F32) \
                + brd(jnp.dot(si, wssi[...], preferred_element_type=F32))
    else:
        web = _b16(whe[...])
        hx = _b16(evx) * web
        hy = _b16(evy) * web
        hz = _b16(evz) * web

    vn = jnp.sqrt(hx * hx + hy * hy + hz * hz + EPS)
    ms = ms + jnp.dot(vn, wsvn0[...], preferred_element_type=F32)
    ms = jnp.maximum(ms, 0.0)
    ux = jnp.dot(hx, wv0[...], preferred_element_type=F32)
    uy = jnp.dot(hy, wv0[...], preferred_element_type=F32)
    uz = jnp.dot(hz, wv0[...], preferred_element_type=F32)
    nrm = jnp.sqrt(ux * ux + uy * uy + uz * uz + EPS)
    g = jax.nn.sigmoid(nrm)
    ux, uy, uz = ux * g, uy * g, uz * g

    ms, ux, uy, uz = _gvp(ms, ux, uy, uz, wh1[...], wss1[...], wsvn1[...],
                          bs1[...], wv1[...], act=True)
    ms, ux, uy, uz = _gvp(ms, ux, uy, uz, wh2[...], wss2[...], wsvn2[...],
                          bs2[...], wv2[...], act=False)

    def mean_k(a):
        w = a.shape[1]
        return jnp.mean(a.reshape(nblk, KNN, w), axis=1)

    def mean_k_lin(a):
        w = a.shape[1]
        a3 = a.reshape(nblk, KNN, w)
        acc = a3[:, 0, :]
        for k in range(1, KNN):
            acc = acc + a3[:, k, :]
        return acc / 16.0

    out_ref[:, 0:HS] = mean_k(ms)
    out_ref[:, HS:HS + HV] = mean_k_lin(ux)
    out_ref[:, HS + HV:HS + 2 * HV] = mean_k_lin(uy)
    out_ref[:, HS + 2 * HV:CVEC] = mean_k_lin(uz)
    out_ref[:, CVEC:] = jnp.zeros((nblk, CPAD - CVEC), F32)


def _full_spec(arr):
    return pl.BlockSpec(arr.shape, lambda b, i: (0,) * arr.ndim)


def _edge_call(dxe, dye, dze, svi, svj, w):
    B, NK, _ = dxe.shape
    N = NK // KNN
    grid = (B, N // NBLK)
    nbk = NBLK * KNN
    e_spec = pl.BlockSpec((1, nbk, 1), lambda b, i: (b, i, 0))
    node_spec = pl.BlockSpec((NBLK, CPAD),
                             lambda b, i, n=N: (b * (n // NBLK) + i, 0))
    edge_spec = pl.BlockSpec((nbk, CPAD),
                             lambda b, i, n=N: (b * (n // NBLK) + i, 0))
    has_node = svi is not None
    args = [dxe, dye, dze]
    specs = [e_spec, e_spec, e_spec]
    if has_node:
        args += [svi, svj]
        specs += [node_spec, edge_spec]
    args += list(w)
    specs += [_full_spec(a) for a in w]
    body = functools.partial(_edge_body, has_node, NBLK)

    def wrapped(*refs):
        body(refs)

    return pl.pallas_call(
        wrapped,
        grid=grid,
        in_specs=specs,
        out_specs=node_spec,
        out_shape=jax.ShapeDtypeStruct((B * N, CPAD), F32),
    )(*args)


def _head_body(sv_ref, wh_ref, wss_ref, wsvn_ref, bs_ref, wv_ref,
               ys_ref, r_ref):
    sv = sv_ref[...]
    s = sv[:, 0:HS]
    vx = sv[:, HS:HS + HV]
    vy = sv[:, HS + HV:HS + 2 * HV]
    vz = sv[:, HS + 2 * HV:CVEC]
    wh = wh_ref[...]
    hx = jnp.dot(vx, wh, preferred_element_type=F32)
    hy = jnp.dot(vy, wh, preferred_element_type=F32)
    hz = jnp.dot(vz, wh, preferred_element_type=F32)
    vn = jnp.sqrt(hx * hx + hy * hy + hz * hz + EPS)
    ys_ref[...] = (jnp.dot(s, wss_ref[...], preferred_element_type=F32)
                   + jnp.dot(vn, wsvn_ref[...], preferred_element_type=F32)
                   + bs_ref[...])
    wv = wv_ref[...]
    ux = jnp.dot(hx, wv, preferred_element_type=F32)
    uy = jnp.dot(hy, wv, preferred_element_type=F32)
    uz = jnp.dot(hz, wv, preferred_element_type=F32)
    v1x, v2x = ux[:, 0:1], ux[:, 1:2]
    v1y, v2y = uy[:, 0:1], uy[:, 1:2]
    v1z, v2z = uz[:, 0:1], uz[:, 1:2]
    n1 = jnp.sqrt(v1x * v1x + v1y * v1y + v1z * v1z + EPS)
    e1x, e1y, e1z = v1x / n1, v1y / n1, v1z / n1
    dt = e1x * v2x + e1y * v2y + e1z * v2z
    u2x, u2y, u2z = v2x - dt * e1x, v2y - dt * e1y, v2z - dt * e1z
    n2 = jnp.sqrt(u2x * u2x + u2y * u2y + u2z * u2z + EPS)
    e2x, e2y, e2z = u2x / n2, u2y / n2, u2z / n2
    e3x = e1y * e2z - e1z * e2y
    e3y = e1z * e2x - e1x * e2z
    e3z = e1x * e2y - e1y * e2x
    r_ref[:, 0:1] = e1x
    r_ref[:, 1:2] = e2x
    r_ref[:, 2:3] = e3x
    r_ref[:, 3:4] = e1y
    r_ref[:, 4:5] = e2y
    r_ref[:, 5:6] = e3y
    r_ref[:, 6:7] = e1z
    r_ref[:, 7:8] = e2z
    r_ref[:, 8:9] = e3z
    r_ref[:, 9:16] = jnp.zeros_like(sv[:, 0:7])


def _head_call(sv, w):
    M = sv.shape[0]
    grid = (1, M // NBLK)
    node_spec = pl.BlockSpec((NBLK, CPAD), lambda b, i: (i, 0))
    return pl.pallas_call(
        _head_body,
        grid=grid,
        in_specs=[node_spec] + [_full_spec(a) for a in w],
        out_specs=[pl.BlockSpec((NBLK, HS), lambda b, i: (i, 0)),
                   pl.BlockSpec((NBLK, 16), lambda b, i: (i, 0))],
        out_shape=[jax.ShapeDtypeStruct((M, HS), F32),
                   jax.ShapeDtypeStruct((M, 16), F32)],
    )(sv, *w)


def _prep_layer0(prm, has_node):
    wh, ws, bs, wv = prm['Wh'], prm['Ws'], prm['bs'], prm['Wv']
    out = []
    if has_node:
        out += [wh]
    else:
        out += [wh[16:17]]
    if has_node:
        out += [ws[0:128]]
    out += [ws[129:191], ws[191:192]]
    if has_node:
        out += [ws[192:320]]
    out += [ws[320:353], bs[None], wv]
    return out


def _prep_layer(prm):
    return [prm['Wh'], prm['Ws'][0:HS], prm['Ws'][HS:HS + HV],
            prm['bs'][None], prm['Wv']]


def kernel(p, params):
    B, N, _ = p.shape
    pz = jnp.concatenate([p, jnp.zeros((B, N, 5), F32)], axis=-1)
    pT = jnp.transpose(pz, (0, 2, 1))
    p1 = pz.astype(BF16).astype(F32)
    r1 = pz - p1
    p2 = r1.astype(BF16).astype(F32)
    p3 = r1 - p2
    psp = jnp.concatenate([p1, p2, p3], axis=-1)
    idx, dx, dy, dz = _knn(pz, pT, psp)
    idxf = idx.reshape(B * N * KNN)
    dxe = dx.reshape(B, N * KNN, 1)
    dye = dy.reshape(B, N * KNN, 1)
    dze = dz.reshape(B, N * KNN, 1)

    sv = None
    for g in range(3):
        layers = params['gconv%d' % g]
        has_node = g > 0
        w = (_prep_layer0(layers[0], has_node)
             + _prep_layer(layers[1]) + _prep_layer(layers[2]))
        if has_node:
            svj = _sc_gather(sv, idxf)
            sv = _edge_call(dxe, dye, dze, sv, svj, w)
        else:
            sv = _edge_call(dxe, dye, dze, None, None, w)

    ow = params['out']
    wlist = [ow['Wh'], ow['Ws'][0:HS], ow['Ws'][HS:HS + HV],
             ow['bs'][None], ow['Wv']]
    ys, r16 = _head_call(sv, wlist)
    R = r16[:, :9].reshape(B, N, 3, 3)
    return R, ys.reshape(B, N, HS)
